# speedup vs baseline: 2.9139x; 2.9139x over previous
import jax
import jax.numpy as jnp
from jax.experimental import pallas as pl


def _add_kernel(x_ref, p_ref, o_ref):
    o_ref[...] = x_ref[...] + p_ref[...]


def kernel(x, pos_emb):
    B, S, D = x.shape
    SBLK = 512
    return pl.pallas_call(
        _add_kernel,
        grid=(S // SBLK, B),
        in_specs=[
            pl.BlockSpec((1, SBLK, D), lambda s, b: (b, s, 0)),
            pl.BlockSpec((SBLK, D), lambda s, b: (s, 0)),
        ],
        out_specs=pl.BlockSpec((1, SBLK, D), lambda s, b: (b, s, 0)),
        out_shape=jax.ShapeDtypeStruct((B, S, D), x.dtype),
    )(x, pos_emb)


# speedup vs baseline: 3.2586x; 1.1183x over previous
import jax
import jax.numpy as jnp
from jax.experimental import pallas as pl


def _add_kernel(x_ref, p_ref, o_ref):
    o_ref[...] = x_ref[...] + p_ref[...]


def kernel(x, pos_emb):
    B, S, D = x.shape
    SBLK = 1024
    return pl.pallas_call(
        _add_kernel,
        grid=(S // SBLK, B),
        in_specs=[
            pl.BlockSpec((1, SBLK, D), lambda s, b: (b, s, 0)),
            pl.BlockSpec((SBLK, D), lambda s, b: (s, 0)),
        ],
        out_specs=pl.BlockSpec((1, SBLK, D), lambda s, b: (b, s, 0)),
        out_shape=jax.ShapeDtypeStruct((B, S, D), x.dtype),
    )(x, pos_emb)


# speedup vs baseline: 3.4572x; 1.0609x over previous
import jax
import jax.numpy as jnp
from jax.experimental import pallas as pl


def _add_kernel(x_ref, p_ref, o_ref):
    o_ref[...] = x_ref[...] + p_ref[...]


def kernel(x, pos_emb):
    B, S, D = x.shape
    SBLK = 2048
    return pl.pallas_call(
        _add_kernel,
        grid=(S // SBLK, B),
        in_specs=[
            pl.BlockSpec((1, SBLK, D), lambda s, b: (b, s, 0)),
            pl.BlockSpec((SBLK, D), lambda s, b: (s, 0)),
        ],
        out_specs=pl.BlockSpec((1, SBLK, D), lambda s, b: (b, s, 0)),
        out_shape=jax.ShapeDtypeStruct((B, S, D), x.dtype),
    )(x, pos_emb)


# speedup vs baseline: 3.4600x; 1.0008x over previous
import jax
import jax.numpy as jnp
from jax.experimental import pallas as pl
from jax.experimental.pallas import tpu as pltpu


def _add_kernel(x_ref, p_ref, o_ref):
    o_ref[...] = x_ref[...] + p_ref[...]


def kernel(x, pos_emb):
    B, S, D = x.shape
    SBLK = 2048
    return pl.pallas_call(
        _add_kernel,
        grid=(S // SBLK, B),
        in_specs=[
            pl.BlockSpec((1, SBLK, D), lambda s, b: (b, s, 0)),
            pl.BlockSpec((SBLK, D), lambda s, b: (s, 0)),
        ],
        out_specs=pl.BlockSpec((1, SBLK, D), lambda s, b: (b, s, 0)),
        out_shape=jax.ShapeDtypeStruct((B, S, D), x.dtype),
        compiler_params=pltpu.CompilerParams(
            dimension_semantics=("parallel", "parallel"),
        ),
    )(x, pos_emb)
